# baseline (speedup 1.0000x reference)
import functools

import jax
import jax.numpy as jnp
from jax import lax
from jax.experimental import pallas as pl
from jax.experimental.pallas import tpu as pltpu
from jax.experimental.pallas import tpu_sc as plsc

BT = 32
CH = 512


def _router_body(x_ref, gw_ref, p_ref, be_ref, sl_ref, oh_ref, cs_ref):
    T = x_ref.shape[0]
    E = gw_ref.shape[0]
    NBLK = be_ref.shape[0]
    logits = lax.dot_general(
        x_ref[...], gw_ref[...], (((1,), (1,)), ((), ())),
        preferred_element_type=jnp.float32)
    m = jnp.max(logits, axis=1, keepdims=True)
    col = lax.broadcasted_iota(jnp.int32, logits.shape, 1)
    assign = jnp.min(jnp.where(logits >= m, col, E), axis=1, keepdims=True)
    oh_ref[...] = (col == assign).astype(jnp.float32)

    r_i = lax.broadcasted_iota(jnp.int32, (CH, CH), 0)
    c_i = lax.broadcasted_iota(jnp.int32, (CH, CH), 1)
    tril = (c_i < r_i).astype(jnp.float32)

    def chunk(i, tot):
        oh = oh_ref[pl.ds(i * CH, CH), :]
        cs_ref[pl.ds(i * CH, CH), :] = lax.dot_general(
            tril, oh, (((1,), (0,)), ((), ())),
            preferred_element_type=jnp.float32) + tot
        return tot + jnp.sum(oh, axis=0, keepdims=True)

    counts = lax.fori_loop(0, T // CH, chunk, jnp.zeros((1, E), jnp.float32))

    blocks = jnp.floor((counts + (BT - 1)) * (1.0 / BT))
    e_r = lax.broadcasted_iota(jnp.int32, (E, E), 0)
    e_c = lax.broadcasted_iota(jnp.int32, (E, E), 1)
    incl = (e_r <= e_c).astype(jnp.float32)
    cumb = lax.dot_general(blocks, incl, (((1,), (0,)), ((), ())),
                           preferred_element_type=jnp.float32)
    pad_start = (cumb - blocks) * BT

    oh = oh_ref[...]
    rank = jnp.sum(cs_ref[...] * oh, axis=1, keepdims=True)
    p_ref[...] = (rank + jnp.sum(oh * pad_start, axis=1,
                                 keepdims=True)).astype(jnp.int32)

    bb = lax.broadcasted_iota(jnp.int32, (NBLK, E), 0).astype(jnp.float32)
    be = jnp.sum((cumb <= bb).astype(jnp.int32), axis=1, keepdims=True)
    be = jnp.minimum(be, E - 1)
    be_ref[...] = be

    bef = be.astype(jnp.float32)
    chg = jnp.concatenate(
        [jnp.zeros((1, 1), jnp.float32),
         (bef[1:] != bef[:-1]).astype(jnp.float32)], axis=0)
    n_r = lax.broadcasted_iota(jnp.int32, (NBLK, NBLK), 0)
    n_c = lax.broadcasted_iota(jnp.int32, (NBLK, NBLK), 1)
    low = (n_c <= n_r).astype(jnp.float32)
    nch = lax.dot_general(low, chg, (((1,), (0,)), ((), ())),
                          preferred_element_type=jnp.float32)
    sl_ref[...] = (nch - 2.0 * jnp.floor(nch * 0.5)).astype(jnp.int32)


def _mlp_body(be_ref, sl_ref, xs_ref, g_hbm, u_hbm, d_hbm, ys_ref,
              wg, wu, wd, sg, su, sd):
    b = pl.program_id(0)
    nb = pl.num_programs(0)
    e = be_ref[b]
    s = sl_ref[b]

    def start(ee, ss):
        pltpu.make_async_copy(g_hbm.at[ee], wg.at[ss], sg.at[ss]).start()
        pltpu.make_async_copy(u_hbm.at[ee], wu.at[ss], su.at[ss]).start()
        pltpu.make_async_copy(d_hbm.at[ee], wd.at[ss], sd.at[ss]).start()

    @pl.when(b == 0)
    def _():
        start(e, s)

    bn = jnp.minimum(b + 1, nb - 1)
    en = be_ref[bn]
    sn = sl_ref[bn]

    @pl.when(en != e)
    def _():
        start(en, sn)

    changed = jnp.logical_or(b == 0, be_ref[jnp.maximum(b - 1, 0)] != e)

    @pl.when(changed)
    def _():
        pltpu.make_async_copy(g_hbm.at[e], wg.at[s], sg.at[s]).wait()
        pltpu.make_async_copy(u_hbm.at[e], wu.at[s], su.at[s]).wait()
        pltpu.make_async_copy(d_hbm.at[e], wd.at[s], sd.at[s]).wait()

    x = xs_ref[...]
    g = lax.dot_general(x, wg[s], (((1,), (1,)), ((), ())),
                        preferred_element_type=jnp.float32)
    u = lax.dot_general(x, wu[s], (((1,), (1,)), ((), ())),
                        preferred_element_type=jnp.float32)
    h = (g * jax.nn.sigmoid(g)) * u
    ys_ref[...] = lax.dot_general(h, wd[s], (((1,), (1,)), ((), ())),
                                  preferred_element_type=jnp.float32)


def kernel(hidden_states, gate_w, gate_proj, up_proj, down_proj):
    Bt, St, H = hidden_states.shape
    E, I, _ = gate_proj.shape
    T = Bt * St
    NBLK = T // BT + E
    NPAD = NBLK * BT

    info = plsc.get_sparse_core_info()
    NC = info.num_cores
    NW = NC * info.num_subcores
    RPW = T // NW

    x2d = hidden_states.reshape(T, H)

    p_out, be_out, sl_out = pl.pallas_call(
        _router_body,
        out_shape=[jax.ShapeDtypeStruct((T, 1), jnp.int32),
                   jax.ShapeDtypeStruct((NBLK, 1), jnp.int32),
                   jax.ShapeDtypeStruct((NBLK, 1), jnp.int32)],
        scratch_shapes=[pltpu.VMEM((T, E), jnp.float32),
                        pltpu.VMEM((T, E), jnp.float32)],
    )(x2d, gate_w)
    block_expert = be_out[:, 0]
    block_slot = sl_out[:, 0]
    p2 = p_out.reshape(NW, RPW)

    mesh = plsc.VectorSubcoreMesh(core_axis_name="c", subcore_axis_name="s")

    @functools.partial(
        pl.kernel, mesh=mesh,
        out_type=jax.ShapeDtypeStruct((NPAD, H), jnp.float32),
        scratch_types=[pltpu.VMEM((RPW,), jnp.int32),
                       pltpu.VMEM((RPW, H), jnp.float32),
                       pltpu.SemaphoreType.DMA])
    def _dispatch(x_hbm, idx_hbm, xs_hbm, idx_v, rows_v, sem):
        wid = lax.axis_index("s") * NC + lax.axis_index("c")
        pltpu.sync_copy(idx_hbm.at[wid], idx_v)
        pltpu.sync_copy(x_hbm.at[pl.ds(wid * RPW, RPW)], rows_v)
        pltpu.async_copy(rows_v, xs_hbm.at[idx_v], sem).wait()

    xs = _dispatch(x2d, p2)

    grid_spec = pltpu.PrefetchScalarGridSpec(
        num_scalar_prefetch=2,
        grid=(NBLK,),
        in_specs=[
            pl.BlockSpec((BT, H), lambda b, be, sl: (b, 0)),
            pl.BlockSpec(memory_space=pltpu.MemorySpace.HBM),
            pl.BlockSpec(memory_space=pltpu.MemorySpace.HBM),
            pl.BlockSpec(memory_space=pltpu.MemorySpace.HBM),
        ],
        out_specs=pl.BlockSpec((BT, H), lambda b, be, sl: (b, 0)),
        scratch_shapes=[pltpu.VMEM((2, I, H), jnp.float32),
                        pltpu.VMEM((2, I, H), jnp.float32),
                        pltpu.VMEM((2, H, I), jnp.float32),
                        pltpu.SemaphoreType.DMA((2,)),
                        pltpu.SemaphoreType.DMA((2,)),
                        pltpu.SemaphoreType.DMA((2,))],
    )
    ys = pl.pallas_call(
        _mlp_body,
        grid_spec=grid_spec,
        out_shape=jax.ShapeDtypeStruct((NPAD, H), jnp.float32),
    )(block_expert, block_slot, xs, gate_proj, up_proj, down_proj)

    @functools.partial(
        pl.kernel, mesh=mesh,
        out_type=jax.ShapeDtypeStruct((T, H), jnp.float32),
        scratch_types=[pltpu.VMEM((RPW,), jnp.int32),
                       pltpu.VMEM((RPW, H), jnp.float32),
                       pltpu.SemaphoreType.DMA])
    def _combine(ys_hbm, idx_hbm, out_hbm, idx_v, rows_v, sem):
        wid = lax.axis_index("s") * NC + lax.axis_index("c")
        pltpu.sync_copy(idx_hbm.at[wid], idx_v)
        pltpu.async_copy(ys_hbm.at[idx_v], rows_v, sem).wait()
        pltpu.sync_copy(rows_v, out_hbm.at[pl.ds(wid * RPW, RPW)])

    out = _combine(ys, p2)
    return out.reshape(Bt, St, H)


# speedup vs baseline: 1.4968x; 1.4968x over previous
import functools

import jax
import jax.numpy as jnp
from jax import lax
from jax.experimental import pallas as pl
from jax.experimental.pallas import tpu as pltpu
from jax.experimental.pallas import tpu_sc as plsc

BT = 64
CH = 512
NBUF = 4


def _router_body(x_ref, gw_ref, p_ref, be_ref, sl_ref, nx_ref, r0_ref, nr_ref,
                 oh_ref, cs_ref):
    T = x_ref.shape[0]
    E = gw_ref.shape[0]
    NBLK = be_ref.shape[0]
    logits = lax.dot_general(
        x_ref[...], gw_ref[...], (((1,), (1,)), ((), ())),
        preferred_element_type=jnp.float32)
    m = jnp.max(logits, axis=1, keepdims=True)
    col = lax.broadcasted_iota(jnp.int32, logits.shape, 1)
    assign = jnp.min(jnp.where(logits >= m, col, E), axis=1, keepdims=True)
    oh_ref[...] = (col == assign).astype(jnp.float32)

    r_i = lax.broadcasted_iota(jnp.int32, (CH, CH), 0)
    c_i = lax.broadcasted_iota(jnp.int32, (CH, CH), 1)
    tril = (c_i < r_i).astype(jnp.float32)

    def chunk(i, tot):
        oh = oh_ref[pl.ds(i * CH, CH), :]
        cs_ref[pl.ds(i * CH, CH), :] = lax.dot_general(
            tril, oh, (((1,), (0,)), ((), ())),
            preferred_element_type=jnp.float32) + tot
        return tot + jnp.sum(oh, axis=0, keepdims=True)

    counts = lax.fori_loop(0, T // CH, chunk, jnp.zeros((1, E), jnp.float32))

    blocks = jnp.floor((counts + (BT - 1)) * (1.0 / BT))
    e_r = lax.broadcasted_iota(jnp.int32, (E, E), 0)
    e_c = lax.broadcasted_iota(jnp.int32, (E, E), 1)
    incl = (e_r <= e_c).astype(jnp.float32)
    cumb = lax.dot_general(blocks, incl, (((1,), (0,)), ((), ())),
                           preferred_element_type=jnp.float32)
    pad_start = (cumb - blocks) * BT

    oh = oh_ref[...]
    rank = jnp.sum(cs_ref[...] * oh, axis=1, keepdims=True)
    p_ref[...] = (rank + jnp.sum(oh * pad_start, axis=1,
                                 keepdims=True)).astype(jnp.int32)

    nr_ref[...] = cumb[:, E - 1:E].astype(jnp.int32)

    bb = lax.broadcasted_iota(jnp.int32, (NBLK, E), 0).astype(jnp.float32)
    be = jnp.sum((cumb <= bb).astype(jnp.int32), axis=1, keepdims=True)
    be = jnp.minimum(be, E - 1)
    be_ref[...] = be

    bef = be.astype(jnp.float32)
    n_r = lax.broadcasted_iota(jnp.int32, (NBLK, NBLK), 0)
    n_c = lax.broadcasted_iota(jnp.int32, (NBLK, NBLK), 1)
    first = lax.broadcasted_iota(jnp.int32, (NBLK, 1), 0) == 0
    chg = jnp.concatenate(
        [jnp.zeros((1, 1), jnp.float32),
         (bef[1:] != bef[:-1]).astype(jnp.float32)], axis=0)
    fb = jnp.where(first, 1.0, chg)
    low = (n_c <= n_r).astype(jnp.float32)
    rid = lax.dot_general(low, chg, (((1,), (0,)), ((), ())),
                          preferred_element_type=jnp.float32)
    sl_ref[...] = (rid - NBUF * jnp.floor(rid * (1.0 / NBUF))).astype(jnp.int32)

    eye = (n_r == n_c).astype(jnp.float32)
    rid_row = lax.dot_general(rid, eye, (((0,), (0,)), ((), ())),
                              preferred_element_type=jnp.float32)
    fb_row = lax.dot_general(fb, eye, (((0,), (0,)), ((), ())),
                             preferred_element_type=jnp.float32)
    m1 = fb_row * (rid_row == rid + (NBUF - 1)).astype(jnp.float32)
    nx = lax.dot_general(m1, bef, (((1,), (0,)), ((), ())),
                         preferred_element_type=jnp.float32)
    has = lax.dot_general(m1, jnp.ones((NBLK, 1), jnp.float32),
                          (((1,), (0,)), ((), ())),
                          preferred_element_type=jnp.float32)
    nx_ref[...] = jnp.where(has > 0, nx, -1.0).astype(jnp.int32)

    r_i = lax.broadcasted_iota(jnp.int32, (8, NBLK), 0).astype(jnp.float32)
    m2 = fb_row * (rid_row == r_i).astype(jnp.float32)
    r0 = lax.dot_general(m2, bef, (((1,), (0,)), ((), ())),
                         preferred_element_type=jnp.float32)
    has2 = lax.dot_general(m2, jnp.ones((NBLK, 1), jnp.float32),
                           (((1,), (0,)), ((), ())),
                           preferred_element_type=jnp.float32)
    r0_ref[...] = jnp.where(has2 > 0, r0, -1.0).astype(jnp.int32)


def _mlp_body(be_ref, sl_ref, nx_ref, r0_ref, nr_ref, xs_ref,
              g_hbm, u_hbm, d_hbm, ys_ref, wg, wu, wd, sg, su, sd):
    b = pl.program_id(0)
    e = be_ref[b]
    s = sl_ref[b]

    def start(ee, ss):
        pltpu.make_async_copy(g_hbm.at[ee], wg.at[ss], sg.at[ss]).start()
        pltpu.make_async_copy(u_hbm.at[ee], wu.at[ss], su.at[ss]).start()
        pltpu.make_async_copy(d_hbm.at[ee], wd.at[ss], sd.at[ss]).start()

    @pl.when(b == 0)
    def _():
        for r in range(NBUF - 1):
            def _issue(r=r):
                start(r0_ref[r], r)
            pl.when(r0_ref[r] >= 0)(_issue)

    changed = jnp.logical_or(b == 0, be_ref[jnp.maximum(b - 1, 0)] != e)
    nx = nx_ref[b]

    @pl.when(jnp.logical_and(changed, nx >= 0))
    def _():
        sn = jnp.where(s == 0, NBUF - 1, s - 1)
        start(nx, sn)

    @pl.when(changed)
    def _():
        pltpu.make_async_copy(g_hbm.at[e], wg.at[s], sg.at[s]).wait()
        pltpu.make_async_copy(u_hbm.at[e], wu.at[s], su.at[s]).wait()
        pltpu.make_async_copy(d_hbm.at[e], wd.at[s], sd.at[s]).wait()

    @pl.when(b < nr_ref[0])
    def _():
        x = xs_ref[...]
        g = lax.dot_general(x, wg[s], (((1,), (1,)), ((), ())),
                            preferred_element_type=jnp.float32)
        u = lax.dot_general(x, wu[s], (((1,), (1,)), ((), ())),
                            preferred_element_type=jnp.float32)
        h = (g * jax.nn.sigmoid(g)) * u
        ys_ref[...] = lax.dot_general(h, wd[s], (((1,), (1,)), ((), ())),
                                      preferred_element_type=jnp.float32)


def kernel(hidden_states, gate_w, gate_proj, up_proj, down_proj):
    Bt, St, H = hidden_states.shape
    E, I, _ = gate_proj.shape
    T = Bt * St
    NBLK = T // BT + E
    NPAD = NBLK * BT

    info = plsc.get_sparse_core_info()
    NC = info.num_cores
    NW = NC * info.num_subcores
    RPW = T // NW

    x2d = hidden_states.reshape(T, H)

    p_out, be_out, sl_out, nx_out, r0_out, nr_out = pl.pallas_call(
        _router_body,
        out_shape=[jax.ShapeDtypeStruct((T, 1), jnp.int32),
                   jax.ShapeDtypeStruct((NBLK, 1), jnp.int32),
                   jax.ShapeDtypeStruct((NBLK, 1), jnp.int32),
                   jax.ShapeDtypeStruct((NBLK, 1), jnp.int32),
                   jax.ShapeDtypeStruct((8, 1), jnp.int32),
                   jax.ShapeDtypeStruct((1, 1), jnp.int32)],
        scratch_shapes=[pltpu.VMEM((T, E), jnp.float32),
                        pltpu.VMEM((T, E), jnp.float32)],
    )(x2d, gate_w)
    block_expert = be_out[:, 0]
    block_slot = sl_out[:, 0]
    block_next = nx_out[:, 0]
    run0 = r0_out[:, 0]
    nreal = nr_out[:, 0]
    p2 = p_out.reshape(NW, RPW)

    mesh = plsc.VectorSubcoreMesh(core_axis_name="c", subcore_axis_name="s")

    @functools.partial(
        pl.kernel, mesh=mesh,
        out_type=jax.ShapeDtypeStruct((NPAD, H), jnp.float32),
        scratch_types=[pltpu.VMEM((RPW,), jnp.int32),
                       pltpu.VMEM((RPW, H), jnp.float32),
                       pltpu.SemaphoreType.DMA])
    def _dispatch(x_hbm, idx_hbm, xs_hbm, idx_v, rows_v, sem):
        wid = lax.axis_index("s") * NC + lax.axis_index("c")
        pltpu.sync_copy(idx_hbm.at[wid], idx_v)
        pltpu.sync_copy(x_hbm.at[pl.ds(wid * RPW, RPW)], rows_v)
        pltpu.async_copy(rows_v, xs_hbm.at[idx_v], sem).wait()

    xs = _dispatch(x2d, p2)

    grid_spec = pltpu.PrefetchScalarGridSpec(
        num_scalar_prefetch=5,
        grid=(NBLK,),
        in_specs=[
            pl.BlockSpec((BT, H), lambda b, *_: (b, 0)),
            pl.BlockSpec(memory_space=pltpu.MemorySpace.HBM),
            pl.BlockSpec(memory_space=pltpu.MemorySpace.HBM),
            pl.BlockSpec(memory_space=pltpu.MemorySpace.HBM),
        ],
        out_specs=pl.BlockSpec((BT, H), lambda b, *_: (b, 0)),
        scratch_shapes=[pltpu.VMEM((NBUF, I, H), jnp.float32),
                        pltpu.VMEM((NBUF, I, H), jnp.float32),
                        pltpu.VMEM((NBUF, H, I), jnp.float32),
                        pltpu.SemaphoreType.DMA((NBUF,)),
                        pltpu.SemaphoreType.DMA((NBUF,)),
                        pltpu.SemaphoreType.DMA((NBUF,))],
    )
    ys = pl.pallas_call(
        _mlp_body,
        grid_spec=grid_spec,
        out_shape=jax.ShapeDtypeStruct((NPAD, H), jnp.float32),
    )(block_expert, block_slot, block_next, run0, nreal,
      xs, gate_proj, up_proj, down_proj)

    @functools.partial(
        pl.kernel, mesh=mesh,
        out_type=jax.ShapeDtypeStruct((T, H), jnp.float32),
        scratch_types=[pltpu.VMEM((RPW,), jnp.int32),
                       pltpu.VMEM((RPW, H), jnp.float32),
                       pltpu.SemaphoreType.DMA])
    def _combine(ys_hbm, idx_hbm, out_hbm, idx_v, rows_v, sem):
        wid = lax.axis_index("s") * NC + lax.axis_index("c")
        pltpu.sync_copy(idx_hbm.at[wid], idx_v)
        pltpu.async_copy(ys_hbm.at[idx_v], rows_v, sem).wait()
        pltpu.sync_copy(rows_v, out_hbm.at[pl.ds(wid * RPW, RPW)])

    out = _combine(ys, p2)
    return out.reshape(Bt, St, H)
